# baseline (speedup 1.0000x reference)
import functools

import jax
import jax.numpy as jnp
from jax import lax
from jax.experimental import pallas as pl
from jax.experimental.pallas import tpu as pltpu
from jax.experimental.pallas import tpu_sc as plsc

_N = 10000
_E = 320000
_D = 128
_NSC = 2
_NTILE = 16
_NW = _NSC * _NTILE
_EPT = _E // _NW
_CH = 64
_NCH = 160
_EPTP = _NCH * _CH
_DSTPAD = 10232
_NPAD = 10240
_WROWS = _NPAD // 8
_NACC = _NPAD + _WROWS
_RPT = _NACC // _NTILE
_RW = 48


def _norm_rows(v):
    return v / (jnp.sqrt(jnp.sum(v * v, axis=-1, keepdims=True)) + 1e-8)


def _lev_embed(lv, le_ref, wl_ref):
    tab = jnp.dot(le_ref[...], wl_ref[...], preferred_element_type=jnp.float32)
    out = (lv == 0).astype(jnp.float32) * tab[0:1]
    out = out + (lv == 1).astype(jnp.float32) * tab[1:2]
    out = out + (lv == 2).astype(jnp.float32) * tab[2:3]
    return out


def _tc_pre_body(x_ref, lv_ref, wn_ref, ws_ref, le_ref, wl_ref, b_ref,
                 hn_ref, hsp_ref):
    x = x_ref[...]
    h = _norm_rows(x)
    hn_ref[...] = jnp.dot(h, wn_ref[...], preferred_element_type=jnp.float32)
    hsp_ref[...] = (jnp.dot(h, ws_ref[...], preferred_element_type=jnp.float32)
                    + _lev_embed(lv_ref[...], le_ref, wl_ref) + b_ref[...])


def _tc_mid_body(hsp_ref, agg_ref, w_ref, x4_ref, lv_ref, wn_ref, ws_ref,
                 le_ref, wl_ref, b_ref, hn_ref, hsp1_ref):
    agg = agg_ref[0, 0:_N] + agg_ref[1, 0:_N]
    wsum = w_ref[0] + w_ref[1]
    out = hsp_ref[...] + agg / jnp.maximum(wsum, 1e-6)
    out = _norm_rows(out)
    f = _norm_rows(jnp.maximum(out, 0.0))

    def _dot(a, b):
        return jnp.sum(a * b)

    def _isfin(v):
        return jnp.abs(v) < jnp.inf

    x4 = x4_ref[...]
    cr_i = ((_dot(x4[0], x4[2]) * _dot(x4[1], x4[3]))
            / (_dot(x4[0], x4[3]) * _dot(x4[1], x4[2]) + 1e-12))
    f4 = f[0:4]
    cr_c = (((_dot(f4[0], f4[2]) + 1.0) * (_dot(f4[1], f4[3]) + 1.0))
            / ((_dot(f4[0], f4[3]) + 1.0) * (_dot(f4[1], f4[2]) + 1.0) + 1e-12))
    factor = jnp.sqrt(jnp.abs(cr_i / (cr_c + 1e-12)))
    ok = _isfin(cr_i) & _isfin(cr_c) & (cr_c != 0.0) & _isfin(factor)
    factor = jnp.where(ok, factor, 1.0)

    g = f * factor
    hn_ref[...] = jnp.dot(g, wn_ref[...], preferred_element_type=jnp.float32)
    hsp1_ref[...] = (jnp.dot(g, ws_ref[...], preferred_element_type=jnp.float32)
                     + _lev_embed(lv_ref[...], le_ref, wl_ref) + b_ref[...])


def _tc_post_body(hsp_ref, agg_ref, w_ref, out_ref):
    agg = agg_ref[0, 0:_N] + agg_ref[1, 0:_N]
    wsum = w_ref[0] + w_ref[1]
    out = hsp_ref[...] + agg / jnp.maximum(wsum, 1e-6)
    out_ref[...] = _norm_rows(_norm_rows(out))


_f32 = jnp.float32

_tc_pre = pl.pallas_call(
    _tc_pre_body,
    out_shape=[jax.ShapeDtypeStruct((_N, _D), _f32),
               jax.ShapeDtypeStruct((_N, _D), _f32)],
)

_tc_mid = pl.pallas_call(
    _tc_mid_body,
    out_shape=[jax.ShapeDtypeStruct((_N, _D), _f32),
               jax.ShapeDtypeStruct((_N, _D), _f32)],
)

_tc_post = pl.pallas_call(
    _tc_post_body,
    out_shape=jax.ShapeDtypeStruct((_N, _D), _f32),
)


@functools.lru_cache(maxsize=None)
def _make_edge_kernel(compute_w):
    mesh = plsc.VectorSubcoreMesh(core_axis_name="c", subcore_axis_name="s")

    out_type = jax.ShapeDtypeStruct((_NSC, _NACC, _D), _f32)
    scratch = [
        pltpu.VMEM((_CH,), jnp.int32),
        pltpu.VMEM((_CH,), jnp.int32),
        pltpu.VMEM((_CH,), jnp.int32),
        pltpu.VMEM((_CH,), jnp.int32),
        pltpu.VMEM((_CH,), _f32),
        pltpu.VMEM((_CH, _D), _f32),
        pltpu.VMEM_SHARED((_NACC, _D), _f32),
        pltpu.SemaphoreType.DMA,
        pltpu.SemaphoreType.DMA,
    ]
    if compute_w:
        scratch.extend([
            pltpu.VMEM((_CH,), jnp.int32),
            pltpu.VMEM((_CH,), jnp.int32),
            pltpu.VMEM((_CH, _D), _f32),
        ])

    def body(hn_hbm, src_hbm, dst_hbm, lev_hbm, *rest):
        if compute_w:
            (out_agg, src_v, dst_v, lsrc_v, ldst_v, lw_v, rows_v,
             acc_a, sem, sem2, idxw_v, grp_v, lwrow_v) = rest
        else:
            (out_agg, src_v, dst_v, lsrc_v, ldst_v, lw_v, rows_v,
             acc_a, sem, sem2) = rest

        c = lax.axis_index("c")
        s = lax.axis_index("s")
        wid = c * _NTILE + s

        zvec = jnp.zeros((16,), _f32)
        zb_v = rows_v.at[pl.ds(0, _RW)]

        def zero_zb(i, _):
            for r in range(_D // 16):
                rows_v[i, pl.ds(r * 16, 16)] = zvec
            return 0

        lax.fori_loop(0, _RW, zero_zb, 0)

        for i in range(_RPT // _RW):
            rs = s * _RPT + i * _RW
            pltpu.sync_copy(zb_v, acc_a.at[pl.ds(rs, _RW)])
        plsc.subcore_barrier()

        io16 = lax.iota(jnp.int32, 16)

        def chunk(ci, _):
            pltpu.sync_copy(src_hbm.at[wid, ci], src_v)
            pltpu.sync_copy(dst_hbm.at[wid, ci], dst_v)
            cp = pltpu.async_copy(hn_hbm.at[src_v], rows_v, sem)
            cls = pltpu.async_copy(lev_hbm.at[src_v], lsrc_v, sem2)
            cld = pltpu.async_copy(lev_hbm.at[dst_v], ldst_v, sem2)
            cls.wait()
            cld.wait()
            for j in range(_CH // 16):
                ls = lsrc_v[pl.ds(j * 16, 16)]
                ld = ldst_v[pl.ds(j * 16, 16)]
                lw16 = 1.0 / (1.0 + jnp.abs(ls - ld).astype(_f32))
                lw_v[pl.ds(j * 16, 16)] = lw16
                if compute_w:
                    dv = dst_v[pl.ds(j * 16, 16)]
                    idxw_v[pl.ds(j * 16, 16)] = (
                        _NPAD + lax.shift_right_logical(dv, 3))
                    grp_v[pl.ds(j * 16, 16)] = dv & 7
            cp.wait()

            def scale_group(j, _):
                lw16 = lw_v[pl.ds(j * 16, 16)]
                if compute_w:
                    g16 = grp_v[pl.ds(j * 16, 16)]
                for k in range(16):
                    e = j * 16 + k
                    lws = lw16[k]
                    for r in range(_D // 16):
                        rows_v[e, pl.ds(r * 16, 16)] = (
                            rows_v[e, pl.ds(r * 16, 16)] * lws)
                    if compute_w:
                        gk = g16[k]
                        for r in range(_D // 16):
                            lsel = jnp.where(gk == r, lws, 0.0)
                            lwrow_v[e, pl.ds(r * 16, 16)] = jnp.where(
                                io16 == 0, lsel, 0.0)
                return 0

            lax.fori_loop(0, _CH // 16, scale_group, 0)

            pltpu.sync_copy(rows_v, acc_a.at[dst_v], add=True)
            if compute_w:
                pltpu.sync_copy(lwrow_v, acc_a.at[idxw_v], add=True)
            return 0

        lax.fori_loop(0, _NCH, chunk, 0)
        plsc.subcore_barrier()

        for i in range(_RPT // _RW):
            rs = s * _RPT + i * _RW
            pltpu.sync_copy(acc_a.at[pl.ds(rs, _RW)], zb_v)
            pltpu.sync_copy(zb_v, out_agg.at[c, pl.ds(rs, _RW)])

    return pl.kernel(body, mesh=mesh, out_type=out_type, scratch_types=scratch)


def kernel(x, edge_index, node_levels, Ws0, Wn0, Wl0, Le0, b0,
           Ws1, Wn1, Wl1, Le1, b1):
    pad = _EPTP - _EPT
    src = jnp.pad(edge_index[0].reshape(_NW, _EPT), ((0, 0), (0, pad)),
                  constant_values=0).reshape(_NW, _NCH, _CH)
    dst = jnp.pad(edge_index[1].reshape(_NW, _EPT), ((0, 0), (0, pad)),
                  constant_values=_DSTPAD).reshape(_NW, _NCH, _CH)
    lv = node_levels.reshape(_N, 1)
    x4 = x[:4]
    b0r = b0.reshape(1, _D)
    b1r = b1.reshape(1, _D)

    hn0, hsp0 = _tc_pre(x, lv, Wn0, Ws0, Le0, Wl0, b0r)
    agg0 = _make_edge_kernel(True)(hn0, src, dst, node_levels)
    w0 = agg0[:, _NPAD:_NACC, ::16].reshape(_NSC, _NPAD, 1)[:, :_N]
    hn1, hsp1 = _tc_mid(hsp0, agg0, w0, x4, lv, Wn1, Ws1, Le1, Wl1, b1r)
    agg1 = _make_edge_kernel(False)(hn1, src, dst, node_levels)
    return _tc_post(hsp1, agg1, w0)


# speedup vs baseline: 1.6722x; 1.6722x over previous
import functools

import jax
import jax.numpy as jnp
from jax import lax
from jax.experimental import pallas as pl
from jax.experimental.pallas import tpu as pltpu
from jax.experimental.pallas import tpu_sc as plsc

_N = 10000
_E = 320000
_D = 128
_NSC = 2
_NTILE = 16
_NW = _NSC * _NTILE
_EPT = _E // _NW
_CH = 80
_NCH = _EPT // _CH
_NPAD = 10240
_WROWS = _NPAD // 8
_NACC = _NPAD + _WROWS
_RPT = _NACC // _NTILE
_RW = 48


def _norm_rows(v):
    return v / (jnp.sqrt(jnp.sum(v * v, axis=-1, keepdims=True)) + 1e-8)


def _lev_embed(lv, le_ref, wl_ref):
    tab = jnp.dot(le_ref[...], wl_ref[...], preferred_element_type=jnp.float32)
    out = (lv == 0).astype(jnp.float32) * tab[0:1]
    out = out + (lv == 1).astype(jnp.float32) * tab[1:2]
    out = out + (lv == 2).astype(jnp.float32) * tab[2:3]
    return out


def _tc_pre_body(x_ref, lv_ref, wn_ref, ws_ref, le_ref, wl_ref, b_ref,
                 hn_ref, hsp_ref):
    x = x_ref[...]
    h = _norm_rows(x)
    hn_ref[...] = jnp.dot(h, wn_ref[...], preferred_element_type=jnp.float32)
    hsp_ref[...] = (jnp.dot(h, ws_ref[...], preferred_element_type=jnp.float32)
                    + _lev_embed(lv_ref[...], le_ref, wl_ref) + b_ref[...])


def _tc_mid_body(hsp_ref, agg_ref, w_ref, x4_ref, lv_ref, wn_ref, ws_ref,
                 le_ref, wl_ref, b_ref, hn_ref, hsp1_ref):
    agg = agg_ref[0, 0:_N] + agg_ref[1, 0:_N]
    wsum = w_ref[0] + w_ref[1]
    out = hsp_ref[...] + agg / jnp.maximum(wsum, 1e-6)
    out = _norm_rows(out)
    f = _norm_rows(jnp.maximum(out, 0.0))

    def _dot(a, b):
        return jnp.sum(a * b)

    def _isfin(v):
        return jnp.abs(v) < jnp.inf

    x4 = x4_ref[...]
    cr_i = ((_dot(x4[0], x4[2]) * _dot(x4[1], x4[3]))
            / (_dot(x4[0], x4[3]) * _dot(x4[1], x4[2]) + 1e-12))
    f4 = f[0:4]
    cr_c = (((_dot(f4[0], f4[2]) + 1.0) * (_dot(f4[1], f4[3]) + 1.0))
            / ((_dot(f4[0], f4[3]) + 1.0) * (_dot(f4[1], f4[2]) + 1.0) + 1e-12))
    factor = jnp.sqrt(jnp.abs(cr_i / (cr_c + 1e-12)))
    ok = _isfin(cr_i) & _isfin(cr_c) & (cr_c != 0.0) & _isfin(factor)
    factor = jnp.where(ok, factor, 1.0)

    g = f * factor
    hn_ref[...] = jnp.dot(g, wn_ref[...], preferred_element_type=jnp.float32)
    hsp1_ref[...] = (jnp.dot(g, ws_ref[...], preferred_element_type=jnp.float32)
                     + _lev_embed(lv_ref[...], le_ref, wl_ref) + b_ref[...])


def _tc_post_body(hsp_ref, agg_ref, w_ref, out_ref):
    agg = agg_ref[0, 0:_N] + agg_ref[1, 0:_N]
    wsum = w_ref[0] + w_ref[1]
    out = hsp_ref[...] + agg / jnp.maximum(wsum, 1e-6)
    out_ref[...] = _norm_rows(_norm_rows(out))


_f32 = jnp.float32

_tc_pre = pl.pallas_call(
    _tc_pre_body,
    out_shape=[jax.ShapeDtypeStruct((_N, _D), _f32),
               jax.ShapeDtypeStruct((_N, _D), _f32)],
)

_tc_mid = pl.pallas_call(
    _tc_mid_body,
    out_shape=[jax.ShapeDtypeStruct((_N, _D), _f32),
               jax.ShapeDtypeStruct((_N, _D), _f32)],
)

_tc_post = pl.pallas_call(
    _tc_post_body,
    out_shape=jax.ShapeDtypeStruct((_N, _D), _f32),
)


@functools.lru_cache(maxsize=None)
def _make_edge_kernel(compute_w):
    mesh = plsc.VectorSubcoreMesh(core_axis_name="c", subcore_axis_name="s")

    out_type = jax.ShapeDtypeStruct((_NSC, _NACC, _D), _f32)
    scratch = [
        pltpu.VMEM((_CH,), jnp.int32),
        pltpu.VMEM((_CH,), jnp.int32),
        pltpu.VMEM((_CH,), jnp.int32),
        pltpu.VMEM((_CH,), jnp.int32),
        pltpu.VMEM((_CH,), _f32),
        pltpu.VMEM((_CH, _D), _f32),
        pltpu.VMEM((_RW, _D), _f32),
        pltpu.VMEM_SHARED((_NACC, _D), _f32),
        pltpu.SemaphoreType.DMA,
        pltpu.SemaphoreType.DMA,
    ]
    if compute_w:
        scratch.extend([
            pltpu.VMEM((_CH,), jnp.int32),
            pltpu.VMEM((_CH,), jnp.int32),
            pltpu.VMEM((_CH, _D), _f32),
        ])

    def body(hn_hbm, src_hbm, dst_hbm, lev_hbm, *rest):
        if compute_w:
            (out_agg, src_v, dst_v, lsrc_v, ldst_v, lw_v, rows_v, zb_v,
             acc_a, sem, sem2, idxw_v, grp_v, lwrow_v) = rest
        else:
            (out_agg, src_v, dst_v, lsrc_v, ldst_v, lw_v, rows_v, zb_v,
             acc_a, sem, sem2) = rest

        c = lax.axis_index("c")
        s = lax.axis_index("s")
        wid = c * _NTILE + s

        zvec = jnp.zeros((16,), _f32)

        def zero_zb(i, _):
            for r in range(_D // 16):
                zb_v[i, pl.ds(r * 16, 16)] = zvec
            return 0

        lax.fori_loop(0, _RW, zero_zb, 0)

        for i in range(_RPT // _RW):
            rs = s * _RPT + i * _RW
            pltpu.sync_copy(zb_v, acc_a.at[pl.ds(rs, _RW)])
        plsc.subcore_barrier()

        io16 = lax.iota(jnp.int32, 16)

        def chunk(ci, _):
            pltpu.sync_copy(src_hbm.at[wid, ci], src_v)
            pltpu.sync_copy(dst_hbm.at[wid, ci], dst_v)
            cp = pltpu.async_copy(hn_hbm.at[src_v], rows_v, sem)
            cls = pltpu.async_copy(lev_hbm.at[src_v], lsrc_v, sem2)
            cld = pltpu.async_copy(lev_hbm.at[dst_v], ldst_v, sem2)
            cls.wait()
            cld.wait()
            for j in range(_CH // 16):
                ls = lsrc_v[pl.ds(j * 16, 16)]
                ld = ldst_v[pl.ds(j * 16, 16)]
                lw16 = 1.0 / (1.0 + jnp.abs(ls - ld).astype(_f32))
                lw_v[pl.ds(j * 16, 16)] = lw16
                if compute_w:
                    dv = dst_v[pl.ds(j * 16, 16)]
                    idxw_v[pl.ds(j * 16, 16)] = (
                        _NPAD + lax.shift_right_logical(dv, 3))
                    grp_v[pl.ds(j * 16, 16)] = dv & 7
            cp.wait()

            def scale_group(j, _):
                lw16 = lw_v[pl.ds(j * 16, 16)]
                if compute_w:
                    g16 = grp_v[pl.ds(j * 16, 16)]
                for k in range(16):
                    e = j * 16 + k
                    lws = lw16[k]
                    for r in range(_D // 16):
                        rows_v[e, pl.ds(r * 16, 16)] = (
                            rows_v[e, pl.ds(r * 16, 16)] * lws)
                    if compute_w:
                        gk = g16[k]
                        for r in range(_D // 16):
                            lsel = jnp.where(gk == r, lws, 0.0)
                            lwrow_v[e, pl.ds(r * 16, 16)] = jnp.where(
                                io16 == 0, lsel, 0.0)
                return 0

            lax.fori_loop(0, _CH // 16, scale_group, 0)

            pltpu.sync_copy(rows_v, acc_a.at[dst_v], add=True)
            if compute_w:
                pltpu.sync_copy(lwrow_v, acc_a.at[idxw_v], add=True)
            return 0

        lax.fori_loop(0, _NCH, chunk, 0)
        plsc.subcore_barrier()

        for i in range(_RPT // _RW):
            rs = s * _RPT + i * _RW
            pltpu.sync_copy(acc_a.at[pl.ds(rs, _RW)], zb_v)
            pltpu.sync_copy(zb_v, out_agg.at[c, pl.ds(rs, _RW)])

    return pl.kernel(body, mesh=mesh, out_type=out_type, scratch_types=scratch)


def kernel(x, edge_index, node_levels, Ws0, Wn0, Wl0, Le0, b0,
           Ws1, Wn1, Wl1, Le1, b1):
    src = edge_index[0].reshape(_NW, _NCH, _CH)
    dst = edge_index[1].reshape(_NW, _NCH, _CH)
    lv = node_levels.reshape(_N, 1)
    x4 = x[:4]
    b0r = b0.reshape(1, _D)
    b1r = b1.reshape(1, _D)

    hn0, hsp0 = _tc_pre(x, lv, Wn0, Ws0, Le0, Wl0, b0r)
    agg0 = _make_edge_kernel(True)(hn0, src, dst, node_levels)
    w0 = agg0[:, _NPAD:_NACC, ::16].reshape(_NSC, _NPAD, 1)[:, :_N]
    hn1, hsp1 = _tc_mid(hsp0, agg0, w0, x4, lv, Wn1, Ws1, Le1, Wl1, b1r)
    agg1 = _make_edge_kernel(False)(hn1, src, dst, node_levels)
    return _tc_post(hsp1, agg1, w0)
